# Initial kernel scaffold; baseline (speedup 1.0000x reference)
import jax
import jax.numpy as jnp
from jax.experimental import pallas as pl


def kernel(z, edge_index, edge_attr, batch, num_nodes_per_graph, params):
    raise NotImplementedError("write your pallas kernel here")


# speedup vs baseline: 8.7071x; 8.7071x over previous
import functools

import jax
import jax.numpy as jnp
from jax import lax
from jax.experimental import pallas as pl
from jax.experimental.pallas import tpu as pltpu
from jax.experimental.pallas import tpu_sc as plsc

B = 8
NPG = 128
H = 128
EDGES = 32768
EPG = EDGES // B
NN = NPG * NPG
NL = 4

_F32 = jnp.float32


def _perm16(x, idx):
    dnums = lax.GatherDimensionNumbers(
        offset_dims=(), collapsed_slice_dims=(0,), start_index_map=(0,))
    return lax.gather(x, idx[:, None], dnums, (1,),
                      mode=lax.GatherScatterMode.PROMISE_IN_BOUNDS)


def _sc_edge_maps(src, dst, wval):
    mesh = plsc.VectorSubcoreMesh(core_axis_name="c", subcore_axis_name="s")

    @functools.partial(
        pl.kernel,
        out_type=[
            jax.ShapeDtypeStruct((B * NN,), _F32),
            jax.ShapeDtypeStruct((B * NN,), _F32),
            jax.ShapeDtypeStruct((B * NN,), _F32),
        ],
        mesh=mesh,
        compiler_params=pltpu.CompilerParams(needs_layout_passes=False),
        scratch_types=[
            pltpu.VMEM((EPG,), jnp.int32),
            pltpu.VMEM((EPG,), jnp.int32),
            pltpu.VMEM((EPG,), _F32),
            pltpu.VMEM((NN,), _F32),
            pltpu.VMEM((NN,), _F32),
            pltpu.VMEM((NN,), _F32),
        ],
    )
    def k(src_hbm, dst_hbm, w_hbm, cnt_hbm, ex_hbm, wt_hbm,
          src_v, dst_v, w_v, cnt_v, ex_v, wt_v):
        wid = lax.axis_index("s") * 2 + lax.axis_index("c")

        @pl.when(wid < B)
        def _():
            g = wid
            pltpu.sync_copy(src_hbm.at[pl.ds(g * EPG, EPG)], src_v)
            pltpu.sync_copy(dst_hbm.at[pl.ds(g * EPG, EPG)], dst_v)
            pltpu.sync_copy(w_hbm.at[pl.ds(g * EPG, EPG)], w_v)

            z16 = jnp.zeros((16,), _F32)

            def zbody(i, carry):
                cnt_v[pl.ds(i * 16, 16)] = z16
                ex_v[pl.ds(i * 16, 16)] = z16
                wt_v[pl.ds(i * 16, 16)] = z16
                return carry

            lax.fori_loop(0, NN // 16, zbody, 0)

            gbase = g * NPG
            lane = lax.iota(jnp.int32, 16)
            ones16 = jnp.ones((16,), _F32)

            def ebody(i, carry):
                b0 = i * 16
                ls = src_v[pl.ds(b0, 16)] - gbase
                ld = dst_v[pl.ds(b0, 16)] - gbase
                wv = w_v[pl.ds(b0, 16)]
                idx_ds = (ld << 7) + ls
                idx_sd = (ls << 7) + ld
                plsc.addupdate_scatter(cnt_v, [idx_ds], ones16)
                plsc.store_scatter(ex_v, [idx_sd], ones16)
                ckey = (idx_sd << 4) + lane
                skey = lax.sort(ckey, dimension=0, is_stable=False)
                grp = skey >> 4
                nxt = _perm16(grp, (lane + 1) & 15)
                is_last = (grp != nxt) | (lane == 15)
                wvals = _perm16(wv, skey & 15)
                plsc.store_scatter(wt_v, [grp], wvals, mask=is_last)
                return carry

            lax.fori_loop(0, EPG // 16, ebody, 0)

            pltpu.sync_copy(cnt_v, cnt_hbm.at[pl.ds(g * NN, NN)])
            pltpu.sync_copy(ex_v, ex_hbm.at[pl.ds(g * NN, NN)])
            pltpu.sync_copy(wt_v, wt_hbm.at[pl.ds(g * NN, NN)])

    cnt, ex, wt = k(src, dst, wval)
    return (cnt.reshape(B, NPG, NPG), ex.reshape(B, NN), wt.reshape(B, NN))


def _dense_core(z, nq, cmat, p):
    def body(z_r, nq_r, cmat_r, zW1, zb1, zW2, zb2, cW1, cb1, cW2, cb2,
             mpW, mpU1, mpU1b, mpU2, mpU2b, lng, lnb,
             cenW1, cenb1, cenW2, cenb2, matW1, matb1, matW2, matb2,
             ntW1, ntb1, ntW2, ntb2, frW1, frb1, frW2, frb2, eaW1, adjW1,
             h_o, cen_o, mat_o, nt_o, fr_o, p1_o, p2_o, a1_o, b1_o):
        dot = functools.partial(jnp.dot, preferred_element_type=_F32)
        relu = lambda x: jnp.maximum(x, 0.0)
        zv = z_r[...]
        z_cond = dot(relu(dot(zv, zW1[...]) + zb1[...]), zW2[...]) + zb2[...]
        cw = cW1[...]
        t1 = dot(nq_r[...], cw[:H])
        t2 = dot(z_cond, cw[H:])
        pre = relu(t1[None, :, :] + t2[:, None, :] + cb1[...][None])
        h = dot(pre.reshape(B * NPG, H), cW2[...]) + cb2[...]
        cm = cmat_r[...]
        hidot = functools.partial(jnp.dot, preferred_element_type=_F32,
                                  precision=lax.Precision.HIGHEST)
        for l in range(NL):
            msg = dot(h, mpW[l])
            agg = jnp.concatenate(
                [hidot(cm[g], msg[g * NPG:(g + 1) * NPG]) for g in range(B)], axis=0)
            upd = dot(relu(dot(agg, mpU1[l]) + mpU1b[l]), mpU2[l]) + mpU2b[l]
            hr = h + upd
            mu = jnp.mean(hr, axis=-1, keepdims=True)
            var = jnp.mean((hr - mu) ** 2, axis=-1, keepdims=True)
            h = (hr - mu) / jnp.sqrt(var + 1e-5) * lng[l] + lnb[l]
        h_o[...] = h
        cen_o[...] = dot(relu(dot(h, cenW1[...]) + cenb1[...]), cenW2[...]) + cenb2[...]
        mat_o[...] = dot(relu(dot(h, matW1[...]) + matb1[...]), matW2[...]) + matb2[...]
        nt_o[...] = dot(relu(dot(h, ntW1[...]) + ntb1[...]), ntW2[...]) + ntb2[...]
        fr_o[...] = dot(relu(dot(h, frW1[...]) + frb1[...]), frW2[...]) + frb2[...]
        ea = eaW1[...]
        p1_o[...] = dot(h, ea[:H])
        p2_o[...] = dot(h, ea[H:])
        aw = adjW1[...]
        a1_o[...] = dot(h, aw[:H])
        b1_o[...] = dot(h, aw[H:2 * H])

    r2 = lambda a: a.reshape(1, -1)
    out = pl.pallas_call(
        body,
        out_shape=[
            jax.ShapeDtypeStruct((B * NPG, H), _F32),
            jax.ShapeDtypeStruct((B * NPG, 3), _F32),
            jax.ShapeDtypeStruct((B * NPG, 32), _F32),
            jax.ShapeDtypeStruct((B * NPG, 3), _F32),
            jax.ShapeDtypeStruct((B * NPG, 1), _F32),
            jax.ShapeDtypeStruct((B * NPG, H), _F32),
            jax.ShapeDtypeStruct((B * NPG, H), _F32),
            jax.ShapeDtypeStruct((B * NPG, H), _F32),
            jax.ShapeDtypeStruct((B * NPG, H), _F32),
        ],
    )(z, nq, cmat,
      p['z_proj_W1'], r2(p['z_proj_b1']), p['z_proj_W2'], r2(p['z_proj_b2']),
      p['combine_W1'], r2(p['combine_b1']), p['combine_W2'], r2(p['combine_b2']),
      jnp.stack(p['mp_msg_W']), jnp.stack(p['mp_upd_W1']),
      jnp.stack(p['mp_upd_b1']).reshape(NL, 1, H),
      jnp.stack(p['mp_upd_W2']), jnp.stack(p['mp_upd_b2']).reshape(NL, 1, H),
      jnp.stack(p['ln_g']).reshape(NL, 1, H), jnp.stack(p['ln_b']).reshape(NL, 1, H),
      p['cen_W1'], r2(p['cen_b1']), p['cen_W2'], r2(p['cen_b2']),
      p['mat_W1'], r2(p['mat_b1']), p['mat_W2'], r2(p['mat_b2']),
      p['nt_W1'], r2(p['nt_b1']), p['nt_W2'], r2(p['nt_b2']),
      p['fr_W1'], r2(p['fr_b1']), p['fr_W2'], r2(p['fr_b2']),
      p['ea_W1'], p['adj_W1'])
    return out


def _edge_head(src3, dst3, p1, p2, eab1, eaW2, eab2):
    def body(ls_r, ld_r, p1_r, p2_r, b1_r, w2_r, b2_r, out_r):
        g = pl.program_id(0)
        ls = ls_r[0] - g * NPG
        ld = ld_r[0] - g * NPG
        iot = lax.broadcasted_iota(jnp.int32, (NPG, EPG), 0)
        ohs = jnp.where(iot == ls, 1.0, 0.0)
        ohd = jnp.where(iot == ld, 1.0, 0.0)
        cd = (((0,), (0,)), ((), ()))
        g1 = lax.dot_general(ohs, p1_r[...], cd, preferred_element_type=_F32,
                             precision=lax.Precision.HIGHEST)
        g2 = lax.dot_general(ohd, p2_r[...], cd, preferred_element_type=_F32,
                             precision=lax.Precision.HIGHEST)
        act = jnp.maximum(g1 + g2 + b1_r[...], 0.0)
        out_r[...] = jnp.dot(act, w2_r[...], preferred_element_type=_F32) + b2_r[...]

    return pl.pallas_call(
        body,
        grid=(B,),
        in_specs=[
            pl.BlockSpec((1, 1, EPG), lambda g: (g, 0, 0)),
            pl.BlockSpec((1, 1, EPG), lambda g: (g, 0, 0)),
            pl.BlockSpec((NPG, H), lambda g: (g, 0)),
            pl.BlockSpec((NPG, H), lambda g: (g, 0)),
            pl.BlockSpec((1, H), lambda g: (0, 0)),
            pl.BlockSpec((H, 4), lambda g: (0, 0)),
            pl.BlockSpec((1, 4), lambda g: (0, 0)),
        ],
        out_specs=pl.BlockSpec((EPG, 4), lambda g: (g, 0)),
        out_shape=jax.ShapeDtypeStruct((EDGES, 4), _F32),
    )(src3, dst3, p1, p2, eab1, eaW2, eab2)


_BI = 32
_NIB = NPG // _BI


def _adj_head(h, a1, b1h, w1c, ab1, w2, ab2, w3, ab3):
    def body(hi_r, hj_r, a1_r, b1_r, w1c_r, ab1_r, w2_r, ab2_r, w3_r, ab3_r,
             lg_r, wt_r):
        hi = hi_r[...]
        hj = hj_r[...]
        prod = (hi[:, None, :] * hj[None, :, :]).reshape(_BI * NPG, H)
        t3 = jnp.dot(prod, w1c_r[...], preferred_element_type=_F32)
        base = (a1_r[...][:, None, :] + b1_r[...][None, :, :]).reshape(_BI * NPG, H)
        act = jnp.maximum(base + t3 + ab1_r[...], 0.0)
        a2 = jnp.maximum(jnp.dot(act, w2_r[...], preferred_element_type=_F32)
                         + ab2_r[...], 0.0)
        a3 = (a2.astype(jnp.bfloat16).astype(_F32)).reshape(_BI, NPG, H // 2)
        w3v = w3_r[...].astype(jnp.bfloat16).astype(_F32)
        b3v = ab3_r[...]
        lg_r[...] = jnp.sum(a3 * w3v[0][None, None, :], axis=-1) + b3v[0, 0]
        wt_r[...] = jnp.sum(a3 * w3v[1][None, None, :], axis=-1) + b3v[0, 1]

    return pl.pallas_call(
        body,
        grid=(B, _NIB),
        in_specs=[
            pl.BlockSpec((_BI, H), lambda g, i: (g * _NIB + i, 0)),
            pl.BlockSpec((NPG, H), lambda g, i: (g, 0)),
            pl.BlockSpec((_BI, H), lambda g, i: (g * _NIB + i, 0)),
            pl.BlockSpec((NPG, H), lambda g, i: (g, 0)),
            pl.BlockSpec((H, H), lambda g, i: (0, 0)),
            pl.BlockSpec((1, H), lambda g, i: (0, 0)),
            pl.BlockSpec((H, H // 2), lambda g, i: (0, 0)),
            pl.BlockSpec((1, H // 2), lambda g, i: (0, 0)),
            pl.BlockSpec((2, H // 2), lambda g, i: (0, 0)),
            pl.BlockSpec((1, 2), lambda g, i: (0, 0)),
        ],
        out_specs=[
            pl.BlockSpec((_BI, NPG), lambda g, i: (g * _NIB + i, 0)),
            pl.BlockSpec((_BI, NPG), lambda g, i: (g * _NIB + i, 0)),
        ],
        out_shape=[
            jax.ShapeDtypeStruct((B * NPG, NPG), _F32),
            jax.ShapeDtypeStruct((B * NPG, NPG), _F32),
        ],
    )(h, h, a1, b1h, w1c, ab1, w2, ab2, w3, ab3)


def _offdiag(x):
    return (x[:, :NN - 1].reshape(B, NPG - 1, NPG + 1)[:, :, 1:]
            .reshape(-1))


def kernel(z, edge_index, edge_attr, batch, num_nodes_per_graph, params):
    p = params
    src = edge_index[0]
    dst = edge_index[1]
    wval = edge_attr[:, 0]

    cnt, ex, wt = _sc_edge_maps(src, dst, wval)

    (h, cen, mat, nt, fr, p1, p2, a1, b1h) = _dense_core(
        z, p['node_queries'][:NPG], cnt, p)

    pea = _edge_head(
        src.reshape(B, 1, EPG), dst.reshape(B, 1, EPG),
        p1.reshape(B * NPG, H), p2.reshape(B * NPG, H),
        p['ea_b1'].reshape(1, H), p['ea_W2'], p['ea_b2'].reshape(1, 4))

    logits, weights = _adj_head(
        h, a1, b1h, p['adj_W1'][2 * H:],
        p['adj_b1'].reshape(1, H), p['adj_W2'], p['adj_b2'].reshape(1, H // 2),
        p['adj_W3'].T, p['adj_b3'].reshape(1, 2))

    adj_logits = _offdiag(logits.reshape(B, NN))
    adj_weights = _offdiag(weights.reshape(B, NN))
    adj_target_exist = _offdiag(ex)
    adj_target_weight = _offdiag(wt)

    return (cen, mat, nt, fr, pea,
            adj_logits, adj_weights, adj_target_exist, adj_target_weight)


# speedup vs baseline: 9.7979x; 1.1253x over previous
import functools

import jax
import jax.numpy as jnp
from jax import lax
from jax.experimental import pallas as pl
from jax.experimental.pallas import tpu as pltpu
from jax.experimental.pallas import tpu_sc as plsc

B = 8
NPG = 128
H = 128
EDGES = 32768
EPG = EDGES // B
NN = NPG * NPG
NL = 4

_F32 = jnp.float32


def _perm16(x, idx):
    dnums = lax.GatherDimensionNumbers(
        offset_dims=(), collapsed_slice_dims=(0,), start_index_map=(0,))
    return lax.gather(x, idx[:, None], dnums, (1,),
                      mode=lax.GatherScatterMode.PROMISE_IN_BOUNDS)


def _sc_edge_maps(src, dst, wval):
    mesh = plsc.VectorSubcoreMesh(core_axis_name="c", subcore_axis_name="s")

    @functools.partial(
        pl.kernel,
        out_type=[
            jax.ShapeDtypeStruct((B * NN,), _F32),
            jax.ShapeDtypeStruct((B * NN,), _F32),
            jax.ShapeDtypeStruct((B * NN,), _F32),
        ],
        mesh=mesh,
        compiler_params=pltpu.CompilerParams(needs_layout_passes=False),
        scratch_types=[
            pltpu.VMEM((EPG,), jnp.int32),
            pltpu.VMEM((EPG,), jnp.int32),
            pltpu.VMEM((EPG,), _F32),
            pltpu.VMEM((NN,), _F32),
            pltpu.VMEM((NN,), _F32),
            pltpu.VMEM((NN,), _F32),
        ],
    )
    def k(src_hbm, dst_hbm, w_hbm, cnt_hbm, ex_hbm, wt_hbm,
          src_v, dst_v, w_v, cnt_v, ex_v, wt_v):
        wid = lax.axis_index("s") * 2 + lax.axis_index("c")

        @pl.when(wid < B)
        def _():
            g = wid
            pltpu.sync_copy(src_hbm.at[pl.ds(g * EPG, EPG)], src_v)
            pltpu.sync_copy(dst_hbm.at[pl.ds(g * EPG, EPG)], dst_v)
            pltpu.sync_copy(w_hbm.at[pl.ds(g * EPG, EPG)], w_v)

            z16 = jnp.zeros((16,), _F32)

            def zbody(i, carry):
                cnt_v[pl.ds(i * 16, 16)] = z16
                ex_v[pl.ds(i * 16, 16)] = z16
                wt_v[pl.ds(i * 16, 16)] = z16
                return carry

            lax.fori_loop(0, NN // 16, zbody, 0)

            gbase = g * NPG
            lane = lax.iota(jnp.int32, 16)
            ones16 = jnp.ones((16,), _F32)

            def ebody(i, carry):
                b0 = i * 16
                ls = src_v[pl.ds(b0, 16)] - gbase
                ld = dst_v[pl.ds(b0, 16)] - gbase
                wv = w_v[pl.ds(b0, 16)]
                idx_ds = (ld << 7) + ls
                idx_sd = (ls << 7) + ld
                plsc.addupdate_scatter(cnt_v, [idx_ds], ones16)
                plsc.store_scatter(ex_v, [idx_sd], ones16)
                ckey = (idx_sd << 4) + lane
                skey = lax.sort(ckey, dimension=0, is_stable=False)
                grp = skey >> 4
                nxt = _perm16(grp, (lane + 1) & 15)
                is_last = (grp != nxt) | (lane == 15)
                wvals = _perm16(wv, skey & 15)
                plsc.store_scatter(wt_v, [grp], wvals, mask=is_last)
                return carry

            lax.fori_loop(0, EPG // 16, ebody, 0)

            pltpu.sync_copy(cnt_v, cnt_hbm.at[pl.ds(g * NN, NN)])
            pltpu.sync_copy(ex_v, ex_hbm.at[pl.ds(g * NN, NN)])
            pltpu.sync_copy(wt_v, wt_hbm.at[pl.ds(g * NN, NN)])

    cnt, ex, wt = k(src, dst, wval)
    return (cnt.reshape(B, NPG, NPG), ex.reshape(B, NN), wt.reshape(B, NN))


def _dense_core(z, nq, cmat, p):
    def body(z_r, nq_r, cmat_r, zW1, zb1, zW2, zb2, cW1, cb1, cW2, cb2,
             mpW, mpU1, mpU1b, mpU2, mpU2b, lng, lnb,
             cenW1, cenb1, cenW2, cenb2, matW1, matb1, matW2, matb2,
             ntW1, ntb1, ntW2, ntb2, frW1, frb1, frW2, frb2, adjW1,
             h_o, cen_o, mat_o, nt_o, fr_o, a1_o, b1_o):
        dot = functools.partial(jnp.dot, preferred_element_type=_F32)
        relu = lambda x: jnp.maximum(x, 0.0)
        zv = z_r[...]
        z_cond = dot(relu(dot(zv, zW1[...]) + zb1[...]), zW2[...]) + zb2[...]
        cw = cW1[...]
        t1 = dot(nq_r[...], cw[:H])
        t2 = dot(z_cond, cw[H:])
        pre = relu(t1[None, :, :] + t2[:, None, :] + cb1[...][None])
        h = dot(pre.reshape(B * NPG, H), cW2[...]) + cb2[...]
        cm = cmat_r[...]
        hidot = functools.partial(jnp.dot, preferred_element_type=_F32,
                                  precision=lax.Precision.HIGHEST)
        for l in range(NL):
            msg = dot(h, mpW[l])
            agg = jnp.concatenate(
                [hidot(cm[g], msg[g * NPG:(g + 1) * NPG]) for g in range(B)], axis=0)
            upd = dot(relu(dot(agg, mpU1[l]) + mpU1b[l]), mpU2[l]) + mpU2b[l]
            hr = h + upd
            mu = jnp.mean(hr, axis=-1, keepdims=True)
            var = jnp.mean((hr - mu) ** 2, axis=-1, keepdims=True)
            h = (hr - mu) / jnp.sqrt(var + 1e-5) * lng[l] + lnb[l]
        h_o[...] = h
        cen_o[...] = dot(relu(dot(h, cenW1[...]) + cenb1[...]), cenW2[...]) + cenb2[...]
        mat_o[...] = dot(relu(dot(h, matW1[...]) + matb1[...]), matW2[...]) + matb2[...]
        nt_o[...] = dot(relu(dot(h, ntW1[...]) + ntb1[...]), ntW2[...]) + ntb2[...]
        fr_o[...] = dot(relu(dot(h, frW1[...]) + frb1[...]), frW2[...]) + frb2[...]
        aw = adjW1[...]
        a1_o[...] = dot(h, aw[:H])
        b1_o[...] = dot(h, aw[H:2 * H])

    r2 = lambda a: a.reshape(1, -1)
    out = pl.pallas_call(
        body,
        out_shape=[
            jax.ShapeDtypeStruct((B * NPG, H), _F32),
            jax.ShapeDtypeStruct((B * NPG, 3), _F32),
            jax.ShapeDtypeStruct((B * NPG, 32), _F32),
            jax.ShapeDtypeStruct((B * NPG, 3), _F32),
            jax.ShapeDtypeStruct((B * NPG, 1), _F32),
            jax.ShapeDtypeStruct((B * NPG, H), _F32),
            jax.ShapeDtypeStruct((B * NPG, H), _F32),
        ],
    )(z, nq, cmat,
      p['z_proj_W1'], r2(p['z_proj_b1']), p['z_proj_W2'], r2(p['z_proj_b2']),
      p['combine_W1'], r2(p['combine_b1']), p['combine_W2'], r2(p['combine_b2']),
      jnp.stack(p['mp_msg_W']), jnp.stack(p['mp_upd_W1']),
      jnp.stack(p['mp_upd_b1']).reshape(NL, 1, H),
      jnp.stack(p['mp_upd_W2']), jnp.stack(p['mp_upd_b2']).reshape(NL, 1, H),
      jnp.stack(p['ln_g']).reshape(NL, 1, H), jnp.stack(p['ln_b']).reshape(NL, 1, H),
      p['cen_W1'], r2(p['cen_b1']), p['cen_W2'], r2(p['cen_b2']),
      p['mat_W1'], r2(p['mat_b1']), p['mat_W2'], r2(p['mat_b2']),
      p['nt_W1'], r2(p['nt_b1']), p['nt_W2'], r2(p['nt_b2']),
      p['fr_W1'], r2(p['fr_b1']), p['fr_W2'], r2(p['fr_b2']),
      p['adj_W1'])
    return out


def _edge_head(src3, dst3, h, eaW1, eab1, eaW2, eab2):
    def body(ls_r, ld_r, h_r, w1_r, b1_r, w2_r, b2_r, out_r):
        g = pl.program_id(0)
        ls = ls_r[0] - g * NPG
        ld = ld_r[0] - g * NPG
        iot = lax.broadcasted_iota(jnp.int32, (NPG, EPG), 0)
        ohs = jnp.where(iot == ls, 1.0, 0.0)
        ohd = jnp.where(iot == ld, 1.0, 0.0)
        cd = (((0,), (0,)), ((), ()))
        hv = h_r[...]
        hs = lax.dot_general(ohs, hv, cd, preferred_element_type=_F32)
        hd = lax.dot_general(ohd, hv, cd, preferred_element_type=_F32)
        w1 = w1_r[...]
        dot = functools.partial(jnp.dot, preferred_element_type=_F32)
        act = jnp.maximum(dot(hs, w1[:H]) + dot(hd, w1[H:]) + b1_r[...], 0.0)
        out_r[...] = dot(act, w2_r[...]) + b2_r[...]

    return pl.pallas_call(
        body,
        grid=(B,),
        in_specs=[
            pl.BlockSpec((1, 1, EPG), lambda g: (g, 0, 0)),
            pl.BlockSpec((1, 1, EPG), lambda g: (g, 0, 0)),
            pl.BlockSpec((NPG, H), lambda g: (g, 0)),
            pl.BlockSpec((2 * H, H), lambda g: (0, 0)),
            pl.BlockSpec((1, H), lambda g: (0, 0)),
            pl.BlockSpec((H, 4), lambda g: (0, 0)),
            pl.BlockSpec((1, 4), lambda g: (0, 0)),
        ],
        out_specs=pl.BlockSpec((EPG, 4), lambda g: (g, 0)),
        out_shape=jax.ShapeDtypeStruct((EDGES, 4), _F32),
    )(src3, dst3, h, eaW1, eab1, eaW2, eab2)


_BI = 32
_NIB = NPG // _BI


def _adj_head(h, a1, b1h, w1c, ab1, w2, ab2, w3, ab3):
    def body(hi_r, hj_r, a1_r, b1_r, w1c_r, ab1_r, w2_r, ab2_r, w3_r, ab3_r,
             lg_r, wt_r):
        hi = hi_r[...]
        hj = hj_r[...]
        prod = (hi[:, None, :] * hj[None, :, :]).reshape(_BI * NPG, H)
        t3 = jnp.dot(prod, w1c_r[...], preferred_element_type=_F32)
        base = (a1_r[...][:, None, :] + b1_r[...][None, :, :]).reshape(_BI * NPG, H)
        act = jnp.maximum(base + t3 + ab1_r[...], 0.0)
        a2 = jnp.maximum(jnp.dot(act, w2_r[...], preferred_element_type=_F32)
                         + ab2_r[...], 0.0)
        a3 = (a2.astype(jnp.bfloat16).astype(_F32)).reshape(_BI, NPG, H // 2)
        w3v = w3_r[...].astype(jnp.bfloat16).astype(_F32)
        b3v = ab3_r[...]
        lg_r[...] = jnp.sum(a3 * w3v[0][None, None, :], axis=-1) + b3v[0, 0]
        wt_r[...] = jnp.sum(a3 * w3v[1][None, None, :], axis=-1) + b3v[0, 1]

    return pl.pallas_call(
        body,
        grid=(B, _NIB),
        in_specs=[
            pl.BlockSpec((_BI, H), lambda g, i: (g * _NIB + i, 0)),
            pl.BlockSpec((NPG, H), lambda g, i: (g, 0)),
            pl.BlockSpec((_BI, H), lambda g, i: (g * _NIB + i, 0)),
            pl.BlockSpec((NPG, H), lambda g, i: (g, 0)),
            pl.BlockSpec((H, H), lambda g, i: (0, 0)),
            pl.BlockSpec((1, H), lambda g, i: (0, 0)),
            pl.BlockSpec((H, H // 2), lambda g, i: (0, 0)),
            pl.BlockSpec((1, H // 2), lambda g, i: (0, 0)),
            pl.BlockSpec((2, H // 2), lambda g, i: (0, 0)),
            pl.BlockSpec((1, 2), lambda g, i: (0, 0)),
        ],
        out_specs=[
            pl.BlockSpec((_BI, NPG), lambda g, i: (g * _NIB + i, 0)),
            pl.BlockSpec((_BI, NPG), lambda g, i: (g * _NIB + i, 0)),
        ],
        out_shape=[
            jax.ShapeDtypeStruct((B * NPG, NPG), _F32),
            jax.ShapeDtypeStruct((B * NPG, NPG), _F32),
        ],
    )(h, h, a1, b1h, w1c, ab1, w2, ab2, w3, ab3)


def _offdiag(x):
    return (x[:, :NN - 1].reshape(B, NPG - 1, NPG + 1)[:, :, 1:]
            .reshape(-1))


def kernel(z, edge_index, edge_attr, batch, num_nodes_per_graph, params):
    p = params
    src = edge_index[0]
    dst = edge_index[1]
    wval = edge_attr[:, 0]

    cnt, ex, wt = _sc_edge_maps(src, dst, wval)

    (h, cen, mat, nt, fr, a1, b1h) = _dense_core(
        z, p['node_queries'][:NPG], cnt, p)

    pea = _edge_head(
        src.reshape(B, 1, EPG), dst.reshape(B, 1, EPG), h, p['ea_W1'],
        p['ea_b1'].reshape(1, H), p['ea_W2'], p['ea_b2'].reshape(1, 4))

    logits, weights = _adj_head(
        h, a1, b1h, p['adj_W1'][2 * H:],
        p['adj_b1'].reshape(1, H), p['adj_W2'], p['adj_b2'].reshape(1, H // 2),
        p['adj_W3'].T, p['adj_b3'].reshape(1, 2))

    adj_logits = _offdiag(logits.reshape(B, NN))
    adj_weights = _offdiag(weights.reshape(B, NN))
    adj_target_exist = _offdiag(ex)
    adj_target_weight = _offdiag(wt)

    return (cen, mat, nt, fr, pea,
            adj_logits, adj_weights, adj_target_exist, adj_target_weight)
